# Initial kernel scaffold; baseline (speedup 1.0000x reference)
import functools

import jax
import jax.numpy as jnp
from jax import lax
from jax.experimental import pallas as pl
from jax.experimental.pallas import tpu as pltpu
from jax.experimental.pallas import tpu_sc as plsc

_B, _L_SEQ, _D_IN, _D_OUT = 8, 2048, 256, 128
_N_DIST = 512
_T = 8192
_MARGIN = 1.0

_NC, _NS, _LANES = 2, 16, 16
_NW = _NC * _NS
_PER_W = _T // _NW
_CH = 128
_NCHUNK = _PER_W // _CH


def _mm_body(x_ref, w_ref, b_ref, o_ref):
    acc = jnp.dot(x_ref[...], w_ref[...], preferred_element_type=jnp.float32)
    o_ref[...] = jnp.maximum(acc + b_ref[0:1, :], 0.0)


def _project(flat_x, W, b8):
    m = flat_x.shape[0]
    bm = 2048
    return pl.pallas_call(
        _mm_body,
        grid=(m // bm,),
        in_specs=[
            pl.BlockSpec((bm, _D_IN), lambda i: (i, 0)),
            pl.BlockSpec((_D_IN, _D_OUT), lambda i: (0, 0)),
            pl.BlockSpec((8, _D_OUT), lambda i: (0, 0)),
        ],
        out_specs=pl.BlockSpec((bm, _D_OUT), lambda i: (i, 0)),
        out_shape=jax.ShapeDtypeStruct((m, _D_OUT), jnp.float32),
    )(flat_x, W, b8)


def _sc_score(flat, dist_emb, ant_idx, cor_idx, neg_idx, dist_idx):
    mesh = plsc.VectorSubcoreMesh(
        core_axis_name="c", subcore_axis_name="s",
        num_cores=_NC, num_subcores=_NS)

    @functools.partial(
        pl.kernel,
        out_type=jax.ShapeDtypeStruct((_NW, _LANES), jnp.float32),
        mesh=mesh,
        scratch_types=[
            pltpu.VMEM((_CH,), jnp.int32),
            pltpu.VMEM((_CH,), jnp.int32),
            pltpu.VMEM((_CH,), jnp.int32),
            pltpu.VMEM((_CH,), jnp.int32),
            pltpu.VMEM((_CH, _D_OUT), jnp.float32),
            pltpu.VMEM((_CH, _D_OUT), jnp.float32),
            pltpu.VMEM((_CH, _D_OUT), jnp.float32),
            pltpu.VMEM((_CH, _D_OUT), jnp.float32),
            pltpu.VMEM((_LANES,), jnp.float32),
            pltpu.SemaphoreType.DMA,
        ],
    )
    def k(flat_hbm, demb_hbm, ant_hbm, cor_hbm, neg_hbm, dist_hbm, out_hbm,
          ai_v, ci_v, ni_v, di_v, a_v, c_v, n_v, r_v, out_v, sem):
        wid = lax.axis_index("s") * _NC + lax.axis_index("c")

        def chunk_body(kk, acc):
            row = wid * _NCHUNK + kk
            pltpu.sync_copy(ant_hbm.at[row], ai_v)
            pltpu.sync_copy(cor_hbm.at[row], ci_v)
            pltpu.sync_copy(neg_hbm.at[row], ni_v)
            pltpu.sync_copy(dist_hbm.at[row], di_v)
            cp1 = pltpu.async_copy(flat_hbm.at[ai_v], a_v, sem)
            cp2 = pltpu.async_copy(flat_hbm.at[ci_v], c_v, sem)
            cp3 = pltpu.async_copy(flat_hbm.at[ni_v], n_v, sem)
            cp4 = pltpu.async_copy(demb_hbm.at[di_v], r_v, sem)
            cp1.wait()
            cp2.wait()
            cp3.wait()
            cp4.wait()

            def tri(t, a2):
                s = jnp.zeros((_LANES,), jnp.float32)
                for j in range(_D_OUT // _LANES):
                    sl = pl.ds(j * _LANES, _LANES)
                    ar = a_v[t, sl] + r_v[t, sl]
                    s = s + jnp.abs(ar - c_v[t, sl]) - jnp.abs(ar - n_v[t, sl])
                tot = jnp.sum(s)
                return a2 + jnp.maximum(tot + _MARGIN, 0.0)

            return lax.fori_loop(0, _CH, tri, acc)

        acc = lax.fori_loop(0, _NCHUNK, chunk_body, jnp.float32(0.0))
        out_v[...] = jnp.full((_LANES,), acc, jnp.float32)
        pltpu.sync_copy(out_v, out_hbm.at[wid])

    return k(flat, dist_emb, ant_idx, cor_idx, neg_idx, dist_idx)


def kernel(h_x, antecedents, distances, corefs, W, b, dist_emb):
    flat_x = h_x.reshape(_B * _L_SEQ, _D_IN)
    b8 = jnp.broadcast_to(b, (8, _D_OUT))
    flat = _project(flat_x, W, b8)

    neg = jnp.roll(corefs, 1)
    partials = _sc_score(
        flat, dist_emb,
        antecedents.reshape(-1, _CH).astype(jnp.int32),
        corefs.reshape(-1, _CH).astype(jnp.int32),
        neg.reshape(-1, _CH).astype(jnp.int32),
        distances.reshape(-1, _CH).astype(jnp.int32),
    )
    loss = jnp.sum(partials[:, 0]) * (1.0 / _T)
    return loss, flat.reshape(_B, _L_SEQ, _D_OUT)


# speedup vs baseline: 1.5438x; 1.5438x over previous
import functools

import jax
import jax.numpy as jnp
from jax import lax
from jax.experimental import pallas as pl
from jax.experimental.pallas import tpu as pltpu
from jax.experimental.pallas import tpu_sc as plsc

_B, _L_SEQ, _D_IN, _D_OUT = 8, 2048, 256, 128
_N_DIST = 512
_T = 8192
_MARGIN = 1.0

_NC, _NS, _LANES = 2, 16, 16
_NW = _NC * _NS
_PER_W = _T // _NW
_CH = 128
_NCHUNK = _PER_W // _CH


def _mm_body(x_ref, w_ref, b_ref, o_ref):
    acc = jnp.dot(x_ref[...], w_ref[...], preferred_element_type=jnp.float32)
    o_ref[...] = jnp.maximum(acc + b_ref[0:1, :], 0.0)


def _project(flat_x, W, b8):
    m = flat_x.shape[0]
    bm = 2048
    return pl.pallas_call(
        _mm_body,
        grid=(m // bm,),
        in_specs=[
            pl.BlockSpec((bm, _D_IN), lambda i: (i, 0)),
            pl.BlockSpec((_D_IN, _D_OUT), lambda i: (0, 0)),
            pl.BlockSpec((8, _D_OUT), lambda i: (0, 0)),
        ],
        out_specs=pl.BlockSpec((bm, _D_OUT), lambda i: (i, 0)),
        out_shape=jax.ShapeDtypeStruct((m, _D_OUT), jnp.float32),
    )(flat_x, W, b8)


def _sc_score(flat, dist_emb, ant_idx, cor_idx, neg_idx, dist_idx):
    mesh = plsc.VectorSubcoreMesh(
        core_axis_name="c", subcore_axis_name="s",
        num_cores=_NC, num_subcores=_NS)

    @functools.partial(
        pl.kernel,
        out_type=jax.ShapeDtypeStruct((_NW, _LANES), jnp.float32),
        mesh=mesh,
        compiler_params=pltpu.CompilerParams(needs_layout_passes=False),
        scratch_types=[
            pltpu.VMEM((_CH,), jnp.int32),
            pltpu.VMEM((_CH,), jnp.int32),
            pltpu.VMEM((_CH,), jnp.int32),
            pltpu.VMEM((_CH,), jnp.int32),
            pltpu.VMEM((_CH, _D_OUT), jnp.float32),
            pltpu.VMEM((_CH, _D_OUT), jnp.float32),
            pltpu.VMEM((_CH, _D_OUT), jnp.float32),
            pltpu.VMEM((_CH, _D_OUT), jnp.float32),
            pltpu.VMEM((_CH, _LANES), jnp.float32),
            pltpu.VMEM((_LANES,), jnp.float32),
            pltpu.SemaphoreType.DMA,
        ],
    )
    def k(flat_hbm, demb_hbm, ant_hbm, cor_hbm, neg_hbm, dist_hbm, out_hbm,
          ai_v, ci_v, ni_v, di_v, a_v, c_v, n_v, r_v, smat, out_v, sem):
        wid = lax.axis_index("s") * _NC + lax.axis_index("c")
        iota = lax.iota(jnp.int32, _LANES)

        def chunk_body(kk, accv):
            row = wid * _NCHUNK + kk
            pltpu.sync_copy(ant_hbm.at[row], ai_v)
            pltpu.sync_copy(cor_hbm.at[row], ci_v)
            pltpu.sync_copy(neg_hbm.at[row], ni_v)
            pltpu.sync_copy(dist_hbm.at[row], di_v)
            cp1 = pltpu.async_copy(flat_hbm.at[ai_v], a_v, sem)
            cp2 = pltpu.async_copy(flat_hbm.at[ci_v], c_v, sem)
            cp3 = pltpu.async_copy(flat_hbm.at[ni_v], n_v, sem)
            cp4 = pltpu.async_copy(demb_hbm.at[di_v], r_v, sem)
            cp1.wait()
            cp2.wait()
            cp3.wait()
            cp4.wait()

            def tri(t, carry):
                s = jnp.zeros((_LANES,), jnp.float32)
                for j in range(_D_OUT // _LANES):
                    sl = pl.ds(j * _LANES, _LANES)
                    ar = a_v[t, sl] + r_v[t, sl]
                    s = s + jnp.abs(ar - c_v[t, sl]) - jnp.abs(ar - n_v[t, sl])
                smat[t, :] = s
                return carry

            lax.fori_loop(0, _CH, tri, 0)

            def blk(tb, av):
                rows = tb * _LANES + iota
                tot = jnp.zeros((_LANES,), jnp.float32)
                for l in range(_LANES):
                    col = jnp.full((_LANES,), l, jnp.int32)
                    tot = tot + plsc.load_gather(smat, [rows, col])
                return av + jnp.maximum(tot + _MARGIN, 0.0)

            return lax.fori_loop(0, _CH // _LANES, blk, accv)

        accv = lax.fori_loop(0, _NCHUNK, chunk_body,
                             jnp.zeros((_LANES,), jnp.float32))
        out_v[...] = accv
        pltpu.sync_copy(out_v, out_hbm.at[wid])

    return k(flat, dist_emb, ant_idx, cor_idx, neg_idx, dist_idx)


def kernel(h_x, antecedents, distances, corefs, W, b, dist_emb):
    flat_x = h_x.reshape(_B * _L_SEQ, _D_IN)
    b8 = jnp.broadcast_to(b, (8, _D_OUT))
    flat = _project(flat_x, W, b8)

    neg = jnp.roll(corefs, 1)
    partials = _sc_score(
        flat, dist_emb,
        antecedents.reshape(-1, _CH).astype(jnp.int32),
        corefs.reshape(-1, _CH).astype(jnp.int32),
        neg.reshape(-1, _CH).astype(jnp.int32),
        distances.reshape(-1, _CH).astype(jnp.int32),
    )
    loss = jnp.sum(partials) * (1.0 / _T)
    return loss, flat.reshape(_B, _L_SEQ, _D_OUT)
